# baseline (speedup 1.0000x reference)
import jax
import jax.numpy as jnp
from jax.experimental import pallas as pl
from jax.experimental.pallas import tpu as pltpu
from jax.experimental.pallas import tpu_sc as plsc
from jax import lax

BATCH = 16384
N_FIELDS = 26
DIM = 64
NUM = BATCH * N_FIELDS
NUM_EMB = 1000000
SPLIT = 524288
NUM_WORKERS = 32
PER_WORKER = NUM // NUM_WORKERS
CHUNK = 512
N_CHUNKS = PER_WORKER // CHUNK

TBLK = 8192


def _transpose_body(xa_ref, xb_ref, o_ref):
    o_ref[:, 0:DIM] = xa_ref[...].T
    o_ref[:, DIM : 2 * DIM] = xb_ref[...].T


def _row_major_table(table):
    table_t = table.T
    last_blk = (NUM_EMB - 1) // TBLK
    out = pl.pallas_call(
        _transpose_body,
        grid=(SPLIT // TBLK,),
        in_specs=[
            pl.BlockSpec((DIM, TBLK), lambda i: (0, i)),
            pl.BlockSpec(
                (DIM, TBLK),
                lambda i: (0, jnp.minimum(i + SPLIT // TBLK, last_blk)),
            ),
        ],
        out_specs=pl.BlockSpec((TBLK, 2 * DIM), lambda i: (i, 0)),
        out_shape=jax.ShapeDtypeStruct((SPLIT, 2 * DIM), table.dtype),
    )(table_t, table_t)
    return out.reshape(2 * SPLIT, DIM)


def kernel(x_cat, table, offsets):
    adjusted = x_cat + offsets[None, :]
    remapped = jnp.where(
        adjusted < SPLIT, adjusted * 2, (adjusted - SPLIT) * 2 + 1
    )
    perm = (
        remapped.reshape(BATCH // 8, 8, N_FIELDS // 2, 2)
        .transpose(0, 2, 1, 3)
        .reshape(NUM)
    )
    table_rm = _row_major_table(table)
    mesh = plsc.VectorSubcoreMesh(core_axis_name="c", subcore_axis_name="s")

    @pl.kernel(
        out_type=jax.ShapeDtypeStruct((NUM, DIM), table.dtype),
        mesh=mesh,
        scratch_types=[
            pltpu.VMEM((PER_WORKER,), jnp.int32),
            pltpu.VMEM((CHUNK, DIM), jnp.float32),
            pltpu.VMEM((CHUNK, DIM), jnp.float32),
            pltpu.SemaphoreType.DMA,
            pltpu.SemaphoreType.DMA,
        ],
        compiler_params=pltpu.CompilerParams(use_tc_tiling_on_sc=False),
    )
    def gather_kernel(table_hbm, idx_hbm, out_hbm, idx_v, rows0, rows1, sem0, sem1):
        wid = lax.axis_index("s") * 2 + lax.axis_index("c")
        base = wid * PER_WORKER
        pltpu.sync_copy(idx_hbm.at[pl.ds(base, PER_WORKER)], idx_v)

        def gather_chunk(i, buf, sem):
            pltpu.async_copy(
                table_hbm.at[idx_v.at[pl.ds(i * CHUNK, CHUNK)]], buf, sem
            )

        def drain(buf, sem):
            pltpu.make_async_copy(
                table_hbm.at[idx_v.at[pl.ds(0, CHUNK)]], buf, sem
            ).wait()

        def write_chunk(i, buf):
            pltpu.sync_copy(buf, out_hbm.at[pl.ds(base + i * CHUNK, CHUNK)])

        gather_chunk(0, rows0, sem0)

        @pl.loop(0, N_CHUNKS, step=2)
        def _(i):
            gather_chunk(i + 1, rows1, sem1)
            drain(rows0, sem0)
            write_chunk(i, rows0)

            @pl.when(i < N_CHUNKS - 2)
            def _():
                gather_chunk(i + 2, rows0, sem0)

            drain(rows1, sem1)
            write_chunk(i + 1, rows1)

    out = gather_kernel(table_rm, perm)
    return (
        out.reshape(BATCH // 8, N_FIELDS // 2, 8, 2 * DIM)
        .transpose(0, 2, 1, 3)
        .reshape(BATCH, N_FIELDS * DIM)
    )


# speedup vs baseline: 1.5647x; 1.5647x over previous
import jax
import jax.numpy as jnp
from jax.experimental import pallas as pl
from jax.experimental.pallas import tpu as pltpu
from jax.experimental.pallas import tpu_sc as plsc
from jax import lax

BATCH = 16384
N_FIELDS = 26
DIM = 64
NUM = BATCH * N_FIELDS
NUM_EMB = 1000000
SPLIT = 524288
NUM_WORKERS = 32
PER_WORKER = NUM // NUM_WORKERS
CHUNK = 512
N_CHUNKS = PER_WORKER // CHUNK

TBLK = 16384


def _transpose_body(xa_ref, xb_ref, o_ref):
    x = jnp.concatenate([xa_ref[...], xb_ref[...]], axis=0)
    o_ref[...] = x.T


def _row_major_table(table):
    table_t = table.T
    last_blk = (NUM_EMB - 1) // TBLK
    out = pl.pallas_call(
        _transpose_body,
        grid=(SPLIT // TBLK,),
        in_specs=[
            pl.BlockSpec((DIM, TBLK), lambda i: (0, i)),
            pl.BlockSpec(
                (DIM, TBLK),
                lambda i: (0, jnp.minimum(i + SPLIT // TBLK, last_blk)),
            ),
        ],
        out_specs=pl.BlockSpec((TBLK, 2 * DIM), lambda i: (i, 0)),
        out_shape=jax.ShapeDtypeStruct((SPLIT, 2 * DIM), table.dtype),
    )(table_t, table_t)
    return out.reshape(2 * SPLIT, DIM)


def kernel(x_cat, table, offsets):
    adjusted = x_cat + offsets[None, :]
    remapped = jnp.where(
        adjusted < SPLIT, adjusted * 2, (adjusted - SPLIT) * 2 + 1
    )
    table_rm = _row_major_table(table)
    mesh = plsc.VectorSubcoreMesh(core_axis_name="c", subcore_axis_name="s")

    @pl.kernel(
        out_type=jax.ShapeDtypeStruct((NUM, DIM), table.dtype),
        mesh=mesh,
        scratch_types=[
            pltpu.VMEM((PER_WORKER,), jnp.int32),
            pltpu.VMEM((CHUNK, DIM), jnp.float32),
            pltpu.VMEM((CHUNK, DIM), jnp.float32),
            pltpu.SemaphoreType.DMA,
            pltpu.SemaphoreType.DMA,
        ],
        compiler_params=pltpu.CompilerParams(use_tc_tiling_on_sc=False),
    )
    def gather_kernel(table_hbm, idx_hbm, out_hbm, idx_v, rows0, rows1, sem0, sem1):
        wid = lax.axis_index("s") * 2 + lax.axis_index("c")
        base = wid * PER_WORKER
        pltpu.sync_copy(idx_hbm.at[pl.ds(base, PER_WORKER)], idx_v)

        def gather_chunk(i, buf, sem):
            pltpu.async_copy(
                table_hbm.at[idx_v.at[pl.ds(i * CHUNK, CHUNK)]], buf, sem
            )

        def drain(buf, sem):
            pltpu.make_async_copy(
                table_hbm.at[idx_v.at[pl.ds(0, CHUNK)]], buf, sem
            ).wait()

        def write_chunk(i, buf):
            pltpu.sync_copy(buf, out_hbm.at[pl.ds(base + i * CHUNK, CHUNK)])

        gather_chunk(0, rows0, sem0)

        @pl.loop(0, N_CHUNKS, step=2)
        def _(i):
            gather_chunk(i + 1, rows1, sem1)
            drain(rows0, sem0)
            write_chunk(i, rows0)

            @pl.when(i < N_CHUNKS - 2)
            def _():
                gather_chunk(i + 2, rows0, sem0)

            drain(rows1, sem1)
            write_chunk(i + 1, rows1)

    out = gather_kernel(table_rm, remapped.reshape(NUM))
    return out.reshape(BATCH, N_FIELDS * DIM)
